# baseline (speedup 1.0000x reference)
import functools

import jax
import jax.numpy as jnp
from jax.experimental import pallas as pl
from jax.experimental.pallas import tpu as pltpu
from jax.experimental.pallas import tpu_sc as plsc


def _pack_bf16_pair(x):
    k = x.shape[-1] // 2
    r = x.astype(jnp.bfloat16).astype(jnp.float32)
    b = jax.lax.bitcast_convert_type(r, jnp.uint32)
    lo = b[:, :k] >> 16
    hi = b[:, k:] & jnp.uint32(0xFFFF0000)
    return jax.lax.bitcast_convert_type(lo | hi, jnp.int32)


def _unpack_bf16_pair(p):
    b = jax.lax.bitcast_convert_type(p, jnp.uint32)
    lo = jax.lax.bitcast_convert_type(b << 16, jnp.float32)
    hi = jax.lax.bitcast_convert_type(b & jnp.uint32(0xFFFF0000), jnp.float32)
    return jnp.concatenate([lo, hi], axis=-1)


def _project_body(nf_ref, w1a_ref, w1b_ref, b1_ref, u_ref, v_ref):
    nf = nf_ref[...]
    u_ref[...] = jnp.dot(nf, w1a_ref[...], preferred_element_type=jnp.float32)
    v_ref[...] = (
        jnp.dot(nf, w1b_ref[...], preferred_element_type=jnp.float32) + b1_ref[...]
    )


def _mlp_body(s_ref, t_ref, ef_ref, w1c_ref, w2_ref, b2_ref, w3_ref, b3_ref,
              gamma_ref, beta_ref, out_ref):
    ef = ef_ref[...]
    h = s_ref[...] + t_ref[...] + jnp.dot(
        ef, w1c_ref[...], preferred_element_type=jnp.float32)
    h = jnp.maximum(h, 0.0)
    h = jnp.dot(h, w2_ref[...], preferred_element_type=jnp.float32) + b2_ref[...]
    h = jnp.maximum(h, 0.0)
    y = jnp.dot(h, w3_ref[...], preferred_element_type=jnp.float32) + b3_ref[...]
    mu = jnp.mean(y, axis=-1, keepdims=True)
    var = jnp.mean((y - mu) ** 2, axis=-1, keepdims=True)
    y = (y - mu) * jax.lax.rsqrt(var + 1e-5) * gamma_ref[...] + beta_ref[...]
    out_ref[...] = ef + y


def _sc_gather(u, v, src, dst, window):
    e = src.shape[0]
    dn = u.shape[1]
    src2 = src.reshape(1, e)
    dst2 = dst.reshape(1, e)
    mesh = plsc.VectorSubcoreMesh(core_axis_name="core", subcore_axis_name="subcore")

    @functools.partial(
        pl.kernel,
        out_type=[
            jax.ShapeDtypeStruct((e, dn), jnp.float32),
            jax.ShapeDtypeStruct((e, dn), jnp.float32),
        ],
        mesh=mesh,
        scratch_types=[pltpu.SemaphoreType.DMA, pltpu.SemaphoreType.DMA],
    )
    def gather_kernel(u_hbm, v_hbm, src_hbm, dst_hbm, s_hbm, t_hbm, sem_s, sem_t):
        def body(src_vmem, dst_vmem, s_vmem, t_vmem):
            cs = pltpu.async_copy(u_hbm.at[src_vmem.at[0]], s_vmem, sem_s)
            ct = pltpu.async_copy(v_hbm.at[dst_vmem.at[0]], t_vmem, sem_t)
            cs.wait()
            ct.wait()

        pltpu.emit_pipeline(
            body,
            grid=(e // window,),
            in_specs=[
                pl.BlockSpec((1, window), lambda i: (0, i)),
                pl.BlockSpec((1, window), lambda i: (0, i)),
            ],
            out_specs=[
                pl.BlockSpec((window, dn), lambda i: (i, 0)),
                pl.BlockSpec((window, dn), lambda i: (i, 0)),
            ],
            core_axis_name=("core", "subcore"),
            dimension_semantics=(pltpu.PARALLEL,),
        )(src_hbm, dst_hbm, s_hbm, t_hbm)

    return gather_kernel(u, v, src2, dst2)


def kernel(edge_features, node_features, edge_index, W1, b1, W2, b2, W3, b3,
           gamma, beta):
    e, de = edge_features.shape
    n, dn = node_features.shape
    h_dim = W2.shape[0]

    w1a = W1[:dn]
    w1b = W1[dn:2 * dn]
    w1c = W1[2 * dn:]

    u, v = pl.pallas_call(
        _project_body,
        out_shape=[
            jax.ShapeDtypeStruct((n, h_dim), jnp.float32),
            jax.ShapeDtypeStruct((n, h_dim), jnp.float32),
        ],
    )(node_features, w1a, w1b, b1.reshape(1, h_dim))

    nchunk = 4
    ec = e // nchunk
    be = 2000
    src = edge_index[0]
    dst = edge_index[1]

    def mlp_call(s_k, t_k, ef_k):
        return pl.pallas_call(
            _mlp_body,
            grid=(ec // be,),
            in_specs=[
                pl.BlockSpec((be, h_dim), lambda i: (i, 0)),
                pl.BlockSpec((be, h_dim), lambda i: (i, 0)),
                pl.BlockSpec((be, de), lambda i: (i, 0)),
                pl.BlockSpec((de, h_dim), lambda i: (0, 0)),
                pl.BlockSpec((h_dim, h_dim), lambda i: (0, 0)),
                pl.BlockSpec((1, h_dim), lambda i: (0, 0)),
                pl.BlockSpec((h_dim, de), lambda i: (0, 0)),
                pl.BlockSpec((1, de), lambda i: (0, 0)),
                pl.BlockSpec((1, de), lambda i: (0, 0)),
                pl.BlockSpec((1, de), lambda i: (0, 0)),
            ],
            out_specs=pl.BlockSpec((be, de), lambda i: (i, 0)),
            out_shape=jax.ShapeDtypeStruct((ec, de), jnp.float32),
        )(s_k, t_k, ef_k, w1c, W2, b2.reshape(1, h_dim), W3,
          b3.reshape(1, de), gamma.reshape(1, de), beta.reshape(1, de))

    outs = []
    for k in range(nchunk):
        sl = slice(k * ec, (k + 1) * ec)
        s_k, t_k = _sc_gather(u, v, src[sl], dst[sl], window=128)
        outs.append(mlp_call(s_k, t_k, edge_features[sl]))
    return jnp.concatenate(outs, axis=0)


# speedup vs baseline: 1.0611x; 1.0611x over previous
import functools

import jax
import jax.numpy as jnp
from jax.experimental import pallas as pl
from jax.experimental.pallas import tpu as pltpu
from jax.experimental.pallas import tpu_sc as plsc


def _pack_bf16_pair(x):
    k = x.shape[-1] // 2
    r = x.astype(jnp.bfloat16).astype(jnp.float32)
    b = jax.lax.bitcast_convert_type(r, jnp.uint32)
    lo = b[:, :k] >> 16
    hi = b[:, k:] & jnp.uint32(0xFFFF0000)
    return jax.lax.bitcast_convert_type(lo | hi, jnp.int32)


def _unpack_bf16_pair(p):
    b = jax.lax.bitcast_convert_type(p, jnp.uint32)
    lo = jax.lax.bitcast_convert_type(b << 16, jnp.float32)
    hi = jax.lax.bitcast_convert_type(b & jnp.uint32(0xFFFF0000), jnp.float32)
    return jnp.concatenate([lo, hi], axis=-1)


def _project_body(nf_ref, w1a_ref, w1b_ref, b1_ref, u_ref, v_ref):
    nf = nf_ref[...]
    u_ref[...] = jnp.dot(nf, w1a_ref[...], preferred_element_type=jnp.float32)
    v_ref[...] = (
        jnp.dot(nf, w1b_ref[...], preferred_element_type=jnp.float32) + b1_ref[...]
    )


def _mlp_body(s_ref, t_ref, ef_ref, w1c_ref, w2_ref, b2_ref, w3_ref, b3_ref,
              gamma_ref, beta_ref, out_ref):
    ef = ef_ref[...]
    h = s_ref[...] + t_ref[...] + jnp.dot(
        ef, w1c_ref[...], preferred_element_type=jnp.float32)
    h = jnp.maximum(h, 0.0)
    h = jnp.dot(h, w2_ref[...], preferred_element_type=jnp.float32) + b2_ref[...]
    h = jnp.maximum(h, 0.0)
    y = jnp.dot(h, w3_ref[...], preferred_element_type=jnp.float32) + b3_ref[...]
    mu = jnp.mean(y, axis=-1, keepdims=True)
    var = jnp.mean((y - mu) ** 2, axis=-1, keepdims=True)
    y = (y - mu) * jax.lax.rsqrt(var + 1e-5) * gamma_ref[...] + beta_ref[...]
    out_ref[...] = ef + y


def _sc_gather(u, v, src, dst, window):
    e = src.shape[0]
    dn = u.shape[1]
    src2 = src.reshape(1, e)
    dst2 = dst.reshape(1, e)
    mesh = plsc.VectorSubcoreMesh(core_axis_name="core", subcore_axis_name="subcore")

    @functools.partial(
        pl.kernel,
        out_type=[
            jax.ShapeDtypeStruct((e, dn), jnp.float32),
            jax.ShapeDtypeStruct((e, dn), jnp.float32),
        ],
        mesh=mesh,
        scratch_types=[pltpu.SemaphoreType.DMA, pltpu.SemaphoreType.DMA],
        compiler_params=pltpu.CompilerParams(use_tc_tiling_on_sc=True),
    )
    def gather_kernel(u_hbm, v_hbm, src_hbm, dst_hbm, s_hbm, t_hbm, sem_s, sem_t):
        def body(src_vmem, dst_vmem, s_vmem, t_vmem):
            cs = pltpu.async_copy(u_hbm.at[src_vmem.at[0]], s_vmem, sem_s)
            ct = pltpu.async_copy(v_hbm.at[dst_vmem.at[0]], t_vmem, sem_t)
            cs.wait()
            ct.wait()

        pltpu.emit_pipeline(
            body,
            grid=(e // window,),
            in_specs=[
                pl.BlockSpec((1, window), lambda i: (0, i)),
                pl.BlockSpec((1, window), lambda i: (0, i)),
            ],
            out_specs=[
                pl.BlockSpec((window, dn), lambda i: (i, 0)),
                pl.BlockSpec((window, dn), lambda i: (i, 0)),
            ],
            core_axis_name=("core", "subcore"),
            dimension_semantics=(pltpu.PARALLEL,),
        )(src_hbm, dst_hbm, s_hbm, t_hbm)

    return gather_kernel(u, v, src2, dst2)


def kernel(edge_features, node_features, edge_index, W1, b1, W2, b2, W3, b3,
           gamma, beta):
    e, de = edge_features.shape
    n, dn = node_features.shape
    h_dim = W2.shape[0]

    w1a = W1[:dn]
    w1b = W1[dn:2 * dn]
    w1c = W1[2 * dn:]

    u, v = pl.pallas_call(
        _project_body,
        out_shape=[
            jax.ShapeDtypeStruct((n, h_dim), jnp.float32),
            jax.ShapeDtypeStruct((n, h_dim), jnp.float32),
        ],
    )(node_features, w1a, w1b, b1.reshape(1, h_dim))

    nchunk = 4
    ec = e // nchunk
    be = 4000
    src = edge_index[0]
    dst = edge_index[1]

    def mlp_call(s_k, t_k, ef_k):
        return pl.pallas_call(
            _mlp_body,
            grid=(ec // be,),
            in_specs=[
                pl.BlockSpec((be, h_dim), lambda i: (i, 0)),
                pl.BlockSpec((be, h_dim), lambda i: (i, 0)),
                pl.BlockSpec((be, de), lambda i: (i, 0)),
                pl.BlockSpec((de, h_dim), lambda i: (0, 0)),
                pl.BlockSpec((h_dim, h_dim), lambda i: (0, 0)),
                pl.BlockSpec((1, h_dim), lambda i: (0, 0)),
                pl.BlockSpec((h_dim, de), lambda i: (0, 0)),
                pl.BlockSpec((1, de), lambda i: (0, 0)),
                pl.BlockSpec((1, de), lambda i: (0, 0)),
                pl.BlockSpec((1, de), lambda i: (0, 0)),
            ],
            out_specs=pl.BlockSpec((be, de), lambda i: (i, 0)),
            out_shape=jax.ShapeDtypeStruct((ec, de), jnp.float32),
        )(s_k, t_k, ef_k, w1c, W2, b2.reshape(1, h_dim), W3,
          b3.reshape(1, de), gamma.reshape(1, de), beta.reshape(1, de))

    outs = []
    for k in range(nchunk):
        sl = slice(k * ec, (k + 1) * ec)
        s_k, t_k = _sc_gather(u, v, src[sl], dst[sl], window=128)
        outs.append(mlp_call(s_k, t_k, edge_features[sl]))
    return jnp.concatenate(outs, axis=0)


# speedup vs baseline: 1.1377x; 1.0722x over previous
import functools

import jax
import jax.numpy as jnp
from jax.experimental import pallas as pl
from jax.experimental.pallas import tpu as pltpu
from jax.experimental.pallas import tpu_sc as plsc


def _project_body(nf_ref, w1a_ref, w1b_ref, b1_ref, u_ref, v_ref):
    nf = nf_ref[...]
    u_ref[...] = jnp.dot(nf, w1a_ref[...], preferred_element_type=jnp.float32)
    v_ref[...] = (
        jnp.dot(nf, w1b_ref[...], preferred_element_type=jnp.float32) + b1_ref[...]
    )


def _mlp_body(s_ref, t_ref, ef_ref, w1c_ref, w2_ref, b2_ref, w3_ref, b3_ref,
              gamma_ref, beta_ref, out_ref):
    ef = ef_ref[...]
    h = s_ref[...] + t_ref[...] + jnp.dot(
        ef, w1c_ref[...], preferred_element_type=jnp.float32)
    h = jnp.maximum(h, 0.0)
    h = jnp.dot(h, w2_ref[...], preferred_element_type=jnp.float32) + b2_ref[...]
    h = jnp.maximum(h, 0.0)
    y = jnp.dot(h, w3_ref[...], preferred_element_type=jnp.float32) + b3_ref[...]
    mu = jnp.mean(y, axis=-1, keepdims=True)
    var = jnp.mean((y - mu) ** 2, axis=-1, keepdims=True)
    y = (y - mu) * jax.lax.rsqrt(var + 1e-5) * gamma_ref[...] + beta_ref[...]
    out_ref[...] = ef + y


def _sc_gather(u, v, src, dst, window):
    e = src.shape[0]
    dn = u.shape[1]
    src2 = src.reshape(1, e)
    dst2 = dst.reshape(1, e)
    mesh = plsc.VectorSubcoreMesh(core_axis_name="core", subcore_axis_name="subcore")

    @functools.partial(
        pl.kernel,
        out_type=[
            jax.ShapeDtypeStruct((e, dn), jnp.float32),
            jax.ShapeDtypeStruct((e, dn), jnp.float32),
        ],
        mesh=mesh,
        scratch_types=[pltpu.SemaphoreType.DMA, pltpu.SemaphoreType.DMA],
        compiler_params=pltpu.CompilerParams(use_tc_tiling_on_sc=True),
    )
    def gather_kernel(u_hbm, v_hbm, src_hbm, dst_hbm, s_hbm, t_hbm, sem_s, sem_t):
        def body(src_vmem, dst_vmem, s_vmem, t_vmem):
            cs = pltpu.async_copy(u_hbm.at[src_vmem.at[0]], s_vmem, sem_s)
            ct = pltpu.async_copy(v_hbm.at[dst_vmem.at[0]], t_vmem, sem_t)
            cs.wait()
            ct.wait()

        pltpu.emit_pipeline(
            body,
            grid=(e // window,),
            in_specs=[
                pl.BlockSpec((1, window), lambda i: (0, i)),
                pl.BlockSpec((1, window), lambda i: (0, i)),
            ],
            out_specs=[
                pl.BlockSpec((window, dn), lambda i: (i, 0)),
                pl.BlockSpec((window, dn), lambda i: (i, 0)),
            ],
            core_axis_name=("core", "subcore"),
            dimension_semantics=(pltpu.PARALLEL,),
        )(src_hbm, dst_hbm, s_hbm, t_hbm)

    return gather_kernel(u, v, src2, dst2)


def kernel(edge_features, node_features, edge_index, W1, b1, W2, b2, W3, b3,
           gamma, beta):
    e, de = edge_features.shape
    n, dn = node_features.shape
    h_dim = W2.shape[0]

    w1a = W1[:dn]
    w1b = W1[dn:2 * dn]
    w1c = W1[2 * dn:]

    u, v = pl.pallas_call(
        _project_body,
        out_shape=[
            jax.ShapeDtypeStruct((n, h_dim), jnp.float32),
            jax.ShapeDtypeStruct((n, h_dim), jnp.float32),
        ],
    )(node_features, w1a, w1b, b1.reshape(1, h_dim))

    be = 4000
    src = edge_index[0]
    dst = edge_index[1]

    def mlp_call(s_k, t_k, ef_k):
        return pl.pallas_call(
            _mlp_body,
            grid=(e // be,),
            in_specs=[
                pl.BlockSpec((be, h_dim), lambda i: (i, 0)),
                pl.BlockSpec((be, h_dim), lambda i: (i, 0)),
                pl.BlockSpec((be, de), lambda i: (i, 0)),
                pl.BlockSpec((de, h_dim), lambda i: (0, 0)),
                pl.BlockSpec((h_dim, h_dim), lambda i: (0, 0)),
                pl.BlockSpec((1, h_dim), lambda i: (0, 0)),
                pl.BlockSpec((h_dim, de), lambda i: (0, 0)),
                pl.BlockSpec((1, de), lambda i: (0, 0)),
                pl.BlockSpec((1, de), lambda i: (0, 0)),
                pl.BlockSpec((1, de), lambda i: (0, 0)),
            ],
            out_specs=pl.BlockSpec((be, de), lambda i: (i, 0)),
            out_shape=jax.ShapeDtypeStruct((e, de), jnp.float32),
        )(s_k, t_k, ef_k, w1c, W2, b2.reshape(1, h_dim), W3,
          b3.reshape(1, de), gamma.reshape(1, de), beta.reshape(1, de))

    s, t = _sc_gather(u, v, src, dst, window=128)
    return mlp_call(s, t, edge_features)
